# Initial kernel scaffold; baseline (speedup 1.0000x reference)
import jax
import jax.numpy as jnp
from jax.experimental import pallas as pl


def kernel(x, adj_t, edge_types, W_rel1, W_root1, b1, W_rel2, W_root2, b2):
    raise NotImplementedError("write your pallas kernel here")


# speedup vs baseline: 3.1929x; 3.1929x over previous
import dataclasses
import functools

import jax
import jax.numpy as jnp
from jax import lax
from jax.experimental import pallas as pl
from jax.experimental.pallas import tpu as pltpu
from jax.experimental.pallas import tpu_sc as plsc

N = 10000
E = 320000
D = 128
R = 4
NB = 40960
NTILES = 32
EPT = E // NTILES
CH = 80
NCH = EPT // CH
RPT = N // 16
ZR = 125

_mesh = plsc.VectorSubcoreMesh(core_axis_name="c", subcore_axis_name="s")

_sc_params = pltpu.CompilerParams()
if "needs_layout_passes" in pltpu.CompilerParams.__dataclass_fields__:
    _sc_params = dataclasses.replace(_sc_params, needs_layout_passes=False)


@functools.partial(
    pl.kernel,
    out_type=jax.ShapeDtypeStruct((NTILES, NB), jnp.float32),
    mesh=_mesh,
    compiler_params=_sc_params,
    scratch_types=[
        pltpu.VMEM((NB,), jnp.float32),
        pltpu.VMEM((CH,), jnp.int32),
    ],
)
def _count_kernel(dstc_hbm, out_hbm, cnt_v, idx_v):
    ci = lax.axis_index("c")
    si = lax.axis_index("s")
    wid = ci * 16 + si
    zeros16 = jnp.zeros((16,), jnp.float32)
    ones16 = jnp.ones((16,), jnp.float32)

    @pl.loop(0, NB, step=16)
    def _(i):
        cnt_v[pl.ds(i, 16)] = zeros16

    ebase = wid * EPT

    @pl.loop(0, NCH)
    def _(cn):
        pltpu.sync_copy(dstc_hbm.at[pl.ds(ebase + cn * CH, CH)], idx_v)
        for b in range(CH // 16):
            iv = idx_v[pl.ds(b * 16, 16)]
            plsc.addupdate_scatter(cnt_v, [iv], ones16)

    pltpu.sync_copy(cnt_v, out_hbm.at[wid])


@functools.partial(
    pl.kernel,
    out_type=jax.ShapeDtypeStruct((2, 16, 5, ZR, D), jnp.float32),
    mesh=_mesh,
    compiler_params=_sc_params,
    scratch_types=[
        pltpu.VMEM((CH,), jnp.float32),
        pltpu.VMEM((CH,), jnp.int32),
        pltpu.VMEM((CH,), jnp.int32),
        pltpu.VMEM((CH,), jnp.int32),
        pltpu.VMEM((CH, D), jnp.float32),
        pltpu.VMEM((ZR, D), jnp.float32),
        pltpu.VMEM_SHARED((N, D), jnp.float32),
        pltpu.SemaphoreType.DMA,
    ],
)
def _scatter_kernel(y_hbm, s_hbm, d_hbm, c_hbm, invc_hbm, out_hbm,
                    scal_v, s_v, d_v, c_v, rows_v, zbuf_v, acc_sh, sem):
    ci = lax.axis_index("c")
    si = lax.axis_index("s")
    wid = ci * 16 + si
    zeros16 = jnp.zeros((16,), jnp.float32)

    @pl.loop(0, ZR)
    def _(r):
        for k in range(D // 16):
            zbuf_v[r, pl.ds(k * 16, 16)] = zeros16

    @pl.loop(0, 5)
    def _(k):
        pltpu.sync_copy(zbuf_v, acc_sh.at[pl.ds(si * RPT + k * ZR, ZR)])

    plsc.subcore_barrier()

    ebase = wid * EPT

    @pl.loop(0, NCH)
    def _(cn):
        base = ebase + cn * CH
        pltpu.sync_copy(s_hbm.at[pl.ds(base, CH)], s_v)
        pltpu.sync_copy(d_hbm.at[pl.ds(base, CH)], d_v)
        pltpu.sync_copy(c_hbm.at[pl.ds(base, CH)], c_v)
        pltpu.async_copy(invc_hbm.at[c_v], scal_v, sem).wait()
        pltpu.async_copy(y_hbm.at[s_v], rows_v, sem).wait()
        for b in range(CH // 16):
            sc = scal_v[pl.ds(b * 16, 16)]
            for j in range(16):
                s1 = lax.broadcast_in_dim(sc[j], (16,), ())
                e = b * 16 + j
                for k in range(D // 16):
                    sl = (e, pl.ds(k * 16, 16))
                    rows_v[sl] = rows_v[sl] * s1
        pltpu.sync_copy(rows_v, acc_sh.at[d_v], add=True)

    plsc.subcore_barrier()

    @pl.loop(0, 5)
    def _(k):
        off = si * RPT + k * ZR
        pltpu.sync_copy(acc_sh.at[pl.ds(off, ZR)],
                        out_hbm.at[ci, si, k])


_BN = 1000


def _mm1_body(x_ref, wr_ref, wo_ref, b_ref, y_ref, r_ref):
    xb = x_ref[...]
    for r in range(R):
        y_ref[r] = jnp.dot(xb, wr_ref[r], preferred_element_type=jnp.float32,
                           precision=lax.Precision.HIGHEST)
    r_ref[...] = jnp.dot(xb, wo_ref[...], preferred_element_type=jnp.float32,
                         precision=lax.Precision.HIGHEST) + b_ref[...]


def _mm1(x, W_rel, W_root, b):
    return pl.pallas_call(
        _mm1_body,
        grid=(N // _BN,),
        in_specs=[
            pl.BlockSpec((_BN, D), lambda i: (i, 0)),
            pl.BlockSpec((R, D, D), lambda i: (0, 0, 0)),
            pl.BlockSpec((D, D), lambda i: (0, 0)),
            pl.BlockSpec((1, D), lambda i: (0, 0)),
        ],
        out_specs=[
            pl.BlockSpec((R, _BN, D), lambda i: (0, i, 0)),
            pl.BlockSpec((_BN, D), lambda i: (i, 0)),
        ],
        out_shape=[
            jax.ShapeDtypeStruct((R, N, D), jnp.float32),
            jax.ShapeDtypeStruct((N, D), jnp.float32),
        ],
    )(x, W_rel, W_root, b)


def _inv_body(cnt_ref, inv_ref):
    s = jnp.sum(cnt_ref[...], axis=0)
    inv_ref[...] = 1.0 / jnp.maximum(s, 1.0)


def _inv(cnt_p):
    return pl.pallas_call(
        _inv_body,
        out_shape=jax.ShapeDtypeStruct((NB // D, D), jnp.float32),
    )(cnt_p.reshape(NTILES, NB // D, D))


def _mm2_body(acc_ref, r1_ref, wr_ref, wo_ref, b_ref, y_ref, r2_ref):
    h = jnp.maximum(acc_ref[0] + acc_ref[1] + r1_ref[...], 0.0)
    for r in range(R):
        y_ref[r] = jnp.dot(h, wr_ref[r], preferred_element_type=jnp.float32,
                           precision=lax.Precision.HIGHEST)
    r2_ref[...] = jnp.dot(h, wo_ref[...], preferred_element_type=jnp.float32,
                          precision=lax.Precision.HIGHEST) + b_ref[...]


def _mm2(acc, root1, W_rel, W_root, b):
    return pl.pallas_call(
        _mm2_body,
        grid=(N // _BN,),
        in_specs=[
            pl.BlockSpec((2, _BN, D), lambda i: (0, i, 0)),
            pl.BlockSpec((_BN, D), lambda i: (i, 0)),
            pl.BlockSpec((R, D, D), lambda i: (0, 0, 0)),
            pl.BlockSpec((D, D), lambda i: (0, 0)),
            pl.BlockSpec((1, D), lambda i: (0, 0)),
        ],
        out_specs=[
            pl.BlockSpec((R, _BN, D), lambda i: (0, i, 0)),
            pl.BlockSpec((_BN, D), lambda i: (i, 0)),
        ],
        out_shape=[
            jax.ShapeDtypeStruct((R, N, D), jnp.float32),
            jax.ShapeDtypeStruct((N, D), jnp.float32),
        ],
    )(acc, root1, W_rel, W_root, b)


def _fin_body(acc_ref, r_ref, o_ref):
    o_ref[...] = acc_ref[0] + acc_ref[1] + r_ref[...]


def _fin(acc, root2):
    return pl.pallas_call(
        _fin_body,
        grid=(N // _BN,),
        in_specs=[
            pl.BlockSpec((2, _BN, D), lambda i: (0, i, 0)),
            pl.BlockSpec((_BN, D), lambda i: (i, 0)),
        ],
        out_specs=pl.BlockSpec((_BN, D), lambda i: (i, 0)),
        out_shape=jax.ShapeDtypeStruct((N, D), jnp.float32),
    )(acc, root2)


@jax.jit
def kernel(x, adj_t, edge_types, W_rel1, W_root1, b1, W_rel2, W_root2, b2):
    src = adj_t[0].astype(jnp.int32)
    dst = adj_t[1].astype(jnp.int32)
    t = edge_types.astype(jnp.int32)
    src_row = t * N + src
    dstc = dst * R + t
    b1r = b1.reshape(1, D)
    b2r = b2.reshape(1, D)

    cnt_p = _count_kernel(dstc)
    invc = _inv(cnt_p).reshape(NB)
    y1, root1 = _mm1(x, W_rel1, W_root1, b1r)
    acc1 = _scatter_kernel(y1.reshape(R * N, D), src_row, dst, dstc,
                           invc).reshape(2, N, D)
    y2, root2 = _mm2(acc1, root1, W_rel2, W_root2, b2r)
    acc2 = _scatter_kernel(y2.reshape(R * N, D), src_row, dst, dstc,
                           invc).reshape(2, N, D)
    return _fin(acc2, root2)


# speedup vs baseline: 4.2820x; 1.3411x over previous
import dataclasses
import functools

import jax
import jax.numpy as jnp
from jax import lax
from jax.experimental import pallas as pl
from jax.experimental.pallas import tpu as pltpu
from jax.experimental.pallas import tpu_sc as plsc

N = 10000
E = 320000
D = 128
R = 4
NB = 40960
NTILES = 32
EPT = E // NTILES
CH = 80
NCH = EPT // CH
RPT = N // 16
ZR = 125

_mesh = plsc.VectorSubcoreMesh(core_axis_name="c", subcore_axis_name="s")

_sc_params = pltpu.CompilerParams()
if "needs_layout_passes" in pltpu.CompilerParams.__dataclass_fields__:
    _sc_params = dataclasses.replace(_sc_params, needs_layout_passes=False)


@functools.partial(
    pl.kernel,
    out_type=jax.ShapeDtypeStruct((NTILES, NB), jnp.float32),
    mesh=_mesh,
    compiler_params=_sc_params,
    scratch_types=[
        pltpu.VMEM((NB,), jnp.float32),
        pltpu.VMEM((EPT,), jnp.int32),
    ],
)
def _count_kernel(dstc_hbm, out_hbm, cnt_v, idx_v):
    ci = lax.axis_index("c")
    si = lax.axis_index("s")
    wid = ci * 16 + si
    zeros16 = jnp.zeros((16,), jnp.float32)
    ones16 = jnp.ones((16,), jnp.float32)

    @pl.loop(0, NB, step=16)
    def _(i):
        cnt_v[pl.ds(i, 16)] = zeros16

    pltpu.sync_copy(dstc_hbm.at[pl.ds(wid * EPT, EPT)], idx_v)

    @pl.loop(0, EPT, step=16)
    def _(i):
        iv = idx_v[pl.ds(i, 16)]
        plsc.addupdate_scatter(cnt_v, [iv], ones16)

    pltpu.sync_copy(cnt_v, out_hbm.at[wid])


@functools.partial(
    pl.kernel,
    out_type=jax.ShapeDtypeStruct((E,), jnp.float32),
    mesh=_mesh,
    compiler_params=_sc_params,
    scratch_types=[
        pltpu.VMEM((NB,), jnp.float32),
        pltpu.VMEM((EPT,), jnp.int32),
        pltpu.VMEM((EPT,), jnp.float32),
    ],
)
def _escale_kernel(dstc_hbm, invc_hbm, out_hbm, invc_v, idx_v, esc_v):
    ci = lax.axis_index("c")
    si = lax.axis_index("s")
    wid = ci * 16 + si
    pltpu.sync_copy(invc_hbm, invc_v)
    pltpu.sync_copy(dstc_hbm.at[pl.ds(wid * EPT, EPT)], idx_v)

    @pl.loop(0, EPT, step=16)
    def _(i):
        iv = idx_v[pl.ds(i, 16)]
        esc_v[pl.ds(i, 16)] = plsc.load_gather(invc_v, [iv])

    pltpu.sync_copy(esc_v, out_hbm.at[pl.ds(wid * EPT, EPT)])


NPH = 5
PE = EPT // NPH
PCH = PE // CH


@functools.partial(
    pl.kernel,
    out_type=jax.ShapeDtypeStruct((2, 16, 5, ZR, D), jnp.float32),
    mesh=_mesh,
    compiler_params=_sc_params,
    scratch_types=[
        pltpu.VMEM((PE,), jnp.int32),
        pltpu.VMEM((PCH, CH), jnp.int32),
        pltpu.VMEM((PE,), jnp.float32),
        pltpu.VMEM((2 * CH, D), jnp.float32),
        pltpu.VMEM_SHARED((N, D), jnp.float32),
        pltpu.SemaphoreType.DMA,
        pltpu.SemaphoreType.DMA,
    ],
)
def _scatter_kernel(y_hbm, s_hbm, d_hbm, e_hbm, out_hbm,
                    s_v, d_v, e_v, rows_v, acc_sh, gsem, csem):
    ci = lax.axis_index("c")
    si = lax.axis_index("s")
    wid = ci * 16 + si
    zeros16 = jnp.zeros((16,), jnp.float32)

    for r in range(2 * CH):
        for k in range(D // 16):
            rows_v[r, pl.ds(k * 16, 16)] = zeros16
    for m in range(4):
        nrow = 160 if m < 3 else RPT - 3 * 160
        pltpu.sync_copy(rows_v.at[pl.ds(0, nrow)],
                        acc_sh.at[pl.ds(si * RPT + m * 160, nrow)])

    plsc.subcore_barrier()

    def gather_wait():
        pltpu.make_async_copy(y_hbm.at[pl.ds(0, CH)],
                              rows_v.at[pl.ds(0, CH)], gsem).wait()

    def scat_wait():
        pltpu.make_async_copy(y_hbm.at[pl.ds(0, CH)],
                              acc_sh.at[pl.ds(0, CH)], csem).wait()

    def _slot_base(j):
        slot, base = (j % 2) * CH, j * CH
        if not isinstance(j, int):
            slot = pl.multiple_of(slot, CH)
            base = pl.multiple_of(base, CH)
        return slot, base

    def issue_gather(j):
        slot, base = _slot_base(j)
        pltpu.async_copy(y_hbm.at[s_v.at[pl.ds(base, CH)]],
                         rows_v.at[pl.ds(slot, CH)], gsem)

    def compute_and_scatter(j):
        slot, base = _slot_base(j)
        for b in range(CH // 16):
            sc = e_v[pl.ds(base + b * 16, 16)]
            for jj in range(16):
                s1 = lax.broadcast_in_dim(sc[jj], (16,), ())
                e = b * 16 + jj
                for k in range(D // 16):
                    sl = (slot + e, pl.ds(k * 16, 16))
                    rows_v[sl] = rows_v[sl] * s1
        pltpu.async_copy(rows_v.at[pl.ds(slot, CH)],
                         acc_sh.at[d_v.at[j]], csem, add=True)

    @pl.loop(0, NPH)
    def _(ph):
        @pl.when(ph > 0)
        def _():
            scat_wait()
            scat_wait()

        ebase = wid * EPT + ph * PE
        pltpu.sync_copy(s_hbm.at[pl.ds(ebase, PE)], s_v)
        pltpu.sync_copy(e_hbm.at[pl.ds(ebase, PE)], e_v)
        pltpu.sync_copy(d_hbm.at[wid * NPH + ph], d_v)

        issue_gather(0)
        issue_gather(1)
        gather_wait()
        compute_and_scatter(0)

        @pl.loop(1, PCH - 1)
        def _(j):
            scat_wait()
            issue_gather(j + 1)
            gather_wait()
            compute_and_scatter(j)

        gather_wait()
        compute_and_scatter(PCH - 1)

    scat_wait()
    scat_wait()
    plsc.subcore_barrier()

    @pl.loop(0, 5)
    def _(k):
        off = si * RPT + k * ZR
        pltpu.sync_copy(acc_sh.at[pl.ds(off, ZR)],
                        out_hbm.at[ci, si, k])


_BN = 1000


def _mm1_body(x_ref, wr_ref, wo_ref, b_ref, y_ref, r_ref):
    xb = x_ref[...]
    for r in range(R):
        y_ref[r] = jnp.dot(xb, wr_ref[r], preferred_element_type=jnp.float32,
                           precision=lax.Precision.HIGHEST)
    r_ref[...] = jnp.dot(xb, wo_ref[...], preferred_element_type=jnp.float32,
                         precision=lax.Precision.HIGHEST) + b_ref[...]


def _mm1(x, W_rel, W_root, b):
    return pl.pallas_call(
        _mm1_body,
        grid=(N // _BN,),
        in_specs=[
            pl.BlockSpec((_BN, D), lambda i: (i, 0)),
            pl.BlockSpec((R, D, D), lambda i: (0, 0, 0)),
            pl.BlockSpec((D, D), lambda i: (0, 0)),
            pl.BlockSpec((1, D), lambda i: (0, 0)),
        ],
        out_specs=[
            pl.BlockSpec((R, _BN, D), lambda i: (0, i, 0)),
            pl.BlockSpec((_BN, D), lambda i: (i, 0)),
        ],
        out_shape=[
            jax.ShapeDtypeStruct((R, N, D), jnp.float32),
            jax.ShapeDtypeStruct((N, D), jnp.float32),
        ],
    )(x, W_rel, W_root, b)


def _inv_body(cnt_ref, inv_ref):
    s = jnp.sum(cnt_ref[...], axis=0)
    inv_ref[...] = 1.0 / jnp.maximum(s, 1.0)


def _inv(cnt_p):
    return pl.pallas_call(
        _inv_body,
        out_shape=jax.ShapeDtypeStruct((NB // D, D), jnp.float32),
    )(cnt_p.reshape(NTILES, NB // D, D))


def _mm2_body(acc_ref, r1_ref, wr_ref, wo_ref, b_ref, y_ref, r2_ref):
    h = jnp.maximum(acc_ref[0] + acc_ref[1] + r1_ref[...], 0.0)
    for r in range(R):
        y_ref[r] = jnp.dot(h, wr_ref[r], preferred_element_type=jnp.float32,
                           precision=lax.Precision.HIGHEST)
    r2_ref[...] = jnp.dot(h, wo_ref[...], preferred_element_type=jnp.float32,
                          precision=lax.Precision.HIGHEST) + b_ref[...]


def _mm2(acc, root1, W_rel, W_root, b):
    return pl.pallas_call(
        _mm2_body,
        grid=(N // _BN,),
        in_specs=[
            pl.BlockSpec((2, _BN, D), lambda i: (0, i, 0)),
            pl.BlockSpec((_BN, D), lambda i: (i, 0)),
            pl.BlockSpec((R, D, D), lambda i: (0, 0, 0)),
            pl.BlockSpec((D, D), lambda i: (0, 0)),
            pl.BlockSpec((1, D), lambda i: (0, 0)),
        ],
        out_specs=[
            pl.BlockSpec((R, _BN, D), lambda i: (0, i, 0)),
            pl.BlockSpec((_BN, D), lambda i: (i, 0)),
        ],
        out_shape=[
            jax.ShapeDtypeStruct((R, N, D), jnp.float32),
            jax.ShapeDtypeStruct((N, D), jnp.float32),
        ],
    )(acc, root1, W_rel, W_root, b)


def _fin_body(acc_ref, r_ref, o_ref):
    o_ref[...] = acc_ref[0] + acc_ref[1] + r_ref[...]


def _fin(acc, root2):
    return pl.pallas_call(
        _fin_body,
        grid=(N // _BN,),
        in_specs=[
            pl.BlockSpec((2, _BN, D), lambda i: (0, i, 0)),
            pl.BlockSpec((_BN, D), lambda i: (i, 0)),
        ],
        out_specs=pl.BlockSpec((_BN, D), lambda i: (i, 0)),
        out_shape=jax.ShapeDtypeStruct((N, D), jnp.float32),
    )(acc, root2)


@jax.jit
def kernel(x, adj_t, edge_types, W_rel1, W_root1, b1, W_rel2, W_root2, b2):
    src = adj_t[0].astype(jnp.int32)
    dst = adj_t[1].astype(jnp.int32)
    t = edge_types.astype(jnp.int32)
    src_row = t * N + src
    dstc = dst * R + t
    b1r = b1.reshape(1, D)
    b2r = b2.reshape(1, D)

    cnt_p = _count_kernel(dstc)
    invc = _inv(cnt_p).reshape(NB)
    escale = _escale_kernel(dstc, invc)
    d3 = dst.reshape(NTILES * NPH, PCH, CH)
    y1, root1 = _mm1(x, W_rel1, W_root1, b1r)
    acc1 = _scatter_kernel(y1.reshape(R * N, D), src_row, d3,
                           escale).reshape(2, N, D)
    y2, root2 = _mm2(acc1, root1, W_rel2, W_root2, b2r)
    acc2 = _scatter_kernel(y2.reshape(R * N, D), src_row, d3,
                           escale).reshape(2, N, D)
    return _fin(acc2, root2)


# speedup vs baseline: 6.5705x; 1.5344x over previous
import dataclasses
import functools

import jax
import jax.numpy as jnp
from jax import lax
from jax.experimental import pallas as pl
from jax.experimental.pallas import tpu as pltpu
from jax.experimental.pallas import tpu_sc as plsc

N = 10000
E = 320000
D = 128
R = 4
NB = 40960
NTILES = 32
EPT = E // NTILES
CH = 80
NCH = EPT // CH
RPT = N // 16
ZR = 125

_mesh = plsc.VectorSubcoreMesh(core_axis_name="c", subcore_axis_name="s")

_sc_params = pltpu.CompilerParams()
if "needs_layout_passes" in pltpu.CompilerParams.__dataclass_fields__:
    _sc_params = dataclasses.replace(_sc_params, needs_layout_passes=False)


@functools.partial(
    pl.kernel,
    out_type=jax.ShapeDtypeStruct((NTILES, NB), jnp.float32),
    mesh=_mesh,
    compiler_params=_sc_params,
    scratch_types=[
        pltpu.VMEM((NB,), jnp.float32),
        pltpu.VMEM((EPT,), jnp.int32),
    ],
)
def _count_kernel(dstc_hbm, out_hbm, cnt_v, idx_v):
    ci = lax.axis_index("c")
    si = lax.axis_index("s")
    wid = ci * 16 + si
    zeros16 = jnp.zeros((16,), jnp.float32)
    ones16 = jnp.ones((16,), jnp.float32)

    @pl.loop(0, NB, step=16)
    def _(i):
        cnt_v[pl.ds(i, 16)] = zeros16

    pltpu.sync_copy(dstc_hbm.at[pl.ds(wid * EPT, EPT)], idx_v)

    @pl.loop(0, EPT, step=16)
    def _(i):
        iv = idx_v[pl.ds(i, 16)]
        plsc.addupdate_scatter(cnt_v, [iv], ones16)

    pltpu.sync_copy(cnt_v, out_hbm.at[wid])


@functools.partial(
    pl.kernel,
    out_type=jax.ShapeDtypeStruct((E,), jnp.float32),
    mesh=_mesh,
    compiler_params=_sc_params,
    scratch_types=[
        pltpu.VMEM((NB,), jnp.float32),
        pltpu.VMEM((EPT,), jnp.int32),
        pltpu.VMEM((EPT,), jnp.float32),
    ],
)
def _escale_kernel(dstc_hbm, invc_hbm, out_hbm, invc_v, idx_v, esc_v):
    ci = lax.axis_index("c")
    si = lax.axis_index("s")
    wid = ci * 16 + si
    pltpu.sync_copy(invc_hbm, invc_v)
    pltpu.sync_copy(dstc_hbm.at[pl.ds(wid * EPT, EPT)], idx_v)

    @pl.loop(0, EPT, step=16)
    def _(i):
        iv = idx_v[pl.ds(i, 16)]
        esc_v[pl.ds(i, 16)] = plsc.load_gather(invc_v, [iv])

    pltpu.sync_copy(esc_v, out_hbm.at[pl.ds(wid * EPT, EPT)])


NPH = 5
PE = EPT // NPH
PCH = PE // CH


@functools.partial(
    pl.kernel,
    out_type=jax.ShapeDtypeStruct((2, 16, 5, ZR, D), jnp.float32),
    mesh=_mesh,
    compiler_params=_sc_params,
    scratch_types=[
        pltpu.VMEM((PE,), jnp.int32),
        pltpu.VMEM((PCH, CH), jnp.int32),
        pltpu.VMEM((PE,), jnp.float32),
        pltpu.VMEM((3 * CH, D), jnp.float32),
        pltpu.VMEM_SHARED((N, D), jnp.float32),
        pltpu.SemaphoreType.DMA,
        pltpu.SemaphoreType.DMA,
        pltpu.SemaphoreType.DMA,
        pltpu.SemaphoreType.DMA,
        pltpu.SemaphoreType.DMA,
        pltpu.SemaphoreType.DMA,
    ],
)
def _scatter_kernel(y_hbm, s_hbm, d_hbm, e_hbm, out_hbm,
                    s_v, d_v, e_v, rows_v, acc_sh,
                    gsem0, gsem1, gsem2, csem0, csem1, csem2):
    gsem = (gsem0, gsem1, gsem2)
    csem = (csem0, csem1, csem2)
    ci = lax.axis_index("c")
    si = lax.axis_index("s")
    wid = ci * 16 + si
    zeros16 = jnp.zeros((16,), jnp.float32)

    for r in range(3 * CH):
        for k in range(D // 16):
            rows_v[r, pl.ds(k * 16, 16)] = zeros16
    for m in range(3):
        nrow = 240 if m < 2 else RPT - 2 * 240
        pltpu.sync_copy(rows_v.at[pl.ds(0, nrow)],
                        acc_sh.at[pl.ds(si * RPT + m * 240, nrow)])

    plsc.subcore_barrier()

    def wait_gather(sl):
        pltpu.make_async_copy(y_hbm.at[s_v.at[pl.ds(0, CH)]],
                              rows_v.at[pl.ds(sl * CH, CH)], gsem[sl]).wait()

    def wait_scat(sl):
        pltpu.make_async_copy(rows_v.at[pl.ds(sl * CH, CH)],
                              acc_sh.at[d_v.at[0]], csem[sl]).wait()

    def _base(j):
        base = j * CH
        return base if isinstance(j, int) else pl.multiple_of(base, CH)

    def issue_gather(j, sl):
        pltpu.async_copy(y_hbm.at[s_v.at[pl.ds(_base(j), CH)]],
                         rows_v.at[pl.ds(sl * CH, CH)], gsem[sl])

    def compute(j, sl):
        for b in range(CH // 16):
            sc = e_v[pl.ds(_base(j) + b * 16, 16)]
            for jj in range(16):
                s1 = lax.broadcast_in_dim(sc[jj], (16,), ())
                row = sl * CH + b * 16 + jj
                for k in range(D // 16):
                    ix = (row, pl.ds(k * 16, 16))
                    rows_v[ix] = rows_v[ix] * s1

    def issue_scatter(j, sl):
        pltpu.async_copy(rows_v.at[pl.ds(sl * CH, CH)],
                         acc_sh.at[d_v.at[j]], csem[sl], add=True)

    @pl.loop(0, NPH)
    def _(ph):
        @pl.when(ph > 0)
        def _():
            wait_scat(0)

        ebase = wid * EPT + ph * PE
        pltpu.sync_copy(s_hbm.at[pl.ds(ebase, PE)], s_v)
        pltpu.sync_copy(e_hbm.at[pl.ds(ebase, PE)], e_v)
        pltpu.sync_copy(d_hbm.at[wid * NPH + ph], d_v)

        issue_gather(0, 0)
        issue_gather(1, 1)
        wait_gather(0)
        compute(0, 0)
        issue_scatter(0, 0)
        issue_gather(2, 2)

        @pl.loop(0, (PCH - 1) // 3)
        def _(g):
            j0 = 1 + 3 * g
            for b in range(3):
                c = j0 + b
                sl = (1 + b) % 3
                s2 = b
                wait_gather(sl)
                compute(c, sl)
                wait_scat(s2)

                @pl.when(c + 2 <= PCH - 1)
                def _():
                    issue_gather(c + 2, s2)

                issue_scatter(c, sl)

    wait_scat(0)
    plsc.subcore_barrier()

    @pl.loop(0, 5)
    def _(k):
        off = si * RPT + k * ZR
        pltpu.sync_copy(acc_sh.at[pl.ds(off, ZR)],
                        out_hbm.at[ci, si, k])


_BN = 1000


def _mm1_body(x_ref, wr_ref, wo_ref, b_ref, y_ref, r_ref):
    xb = x_ref[...]
    for r in range(R):
        y_ref[r] = jnp.dot(xb, wr_ref[r], preferred_element_type=jnp.float32,
                           precision=lax.Precision.HIGHEST)
    r_ref[...] = jnp.dot(xb, wo_ref[...], preferred_element_type=jnp.float32,
                         precision=lax.Precision.HIGHEST) + b_ref[...]


def _mm1(x, W_rel, W_root, b):
    return pl.pallas_call(
        _mm1_body,
        grid=(N // _BN,),
        in_specs=[
            pl.BlockSpec((_BN, D), lambda i: (i, 0)),
            pl.BlockSpec((R, D, D), lambda i: (0, 0, 0)),
            pl.BlockSpec((D, D), lambda i: (0, 0)),
            pl.BlockSpec((1, D), lambda i: (0, 0)),
        ],
        out_specs=[
            pl.BlockSpec((R, _BN, D), lambda i: (0, i, 0)),
            pl.BlockSpec((_BN, D), lambda i: (i, 0)),
        ],
        out_shape=[
            jax.ShapeDtypeStruct((R, N, D), jnp.float32),
            jax.ShapeDtypeStruct((N, D), jnp.float32),
        ],
    )(x, W_rel, W_root, b)


def _inv_body(cnt_ref, inv_ref):
    s = jnp.sum(cnt_ref[...], axis=0)
    inv_ref[...] = 1.0 / jnp.maximum(s, 1.0)


def _inv(cnt_p):
    return pl.pallas_call(
        _inv_body,
        out_shape=jax.ShapeDtypeStruct((NB // D, D), jnp.float32),
    )(cnt_p.reshape(NTILES, NB // D, D))


def _mm2_body(acc_ref, r1_ref, wr_ref, wo_ref, b_ref, y_ref, r2_ref):
    h = jnp.maximum(acc_ref[0] + acc_ref[1] + r1_ref[...], 0.0)
    for r in range(R):
        y_ref[r] = jnp.dot(h, wr_ref[r], preferred_element_type=jnp.float32,
                           precision=lax.Precision.HIGHEST)
    r2_ref[...] = jnp.dot(h, wo_ref[...], preferred_element_type=jnp.float32,
                          precision=lax.Precision.HIGHEST) + b_ref[...]


def _mm2(acc, root1, W_rel, W_root, b):
    return pl.pallas_call(
        _mm2_body,
        grid=(N // _BN,),
        in_specs=[
            pl.BlockSpec((2, _BN, D), lambda i: (0, i, 0)),
            pl.BlockSpec((_BN, D), lambda i: (i, 0)),
            pl.BlockSpec((R, D, D), lambda i: (0, 0, 0)),
            pl.BlockSpec((D, D), lambda i: (0, 0)),
            pl.BlockSpec((1, D), lambda i: (0, 0)),
        ],
        out_specs=[
            pl.BlockSpec((R, _BN, D), lambda i: (0, i, 0)),
            pl.BlockSpec((_BN, D), lambda i: (i, 0)),
        ],
        out_shape=[
            jax.ShapeDtypeStruct((R, N, D), jnp.float32),
            jax.ShapeDtypeStruct((N, D), jnp.float32),
        ],
    )(acc, root1, W_rel, W_root, b)


def _fin_body(acc_ref, r_ref, o_ref):
    o_ref[...] = acc_ref[0] + acc_ref[1] + r_ref[...]


def _fin(acc, root2):
    return pl.pallas_call(
        _fin_body,
        grid=(N // _BN,),
        in_specs=[
            pl.BlockSpec((2, _BN, D), lambda i: (0, i, 0)),
            pl.BlockSpec((_BN, D), lambda i: (i, 0)),
        ],
        out_specs=pl.BlockSpec((_BN, D), lambda i: (i, 0)),
        out_shape=jax.ShapeDtypeStruct((N, D), jnp.float32),
    )(acc, root2)


@jax.jit
def kernel(x, adj_t, edge_types, W_rel1, W_root1, b1, W_rel2, W_root2, b2):
    src = adj_t[0].astype(jnp.int32)
    dst = adj_t[1].astype(jnp.int32)
    t = edge_types.astype(jnp.int32)
    src_row = t * N + src
    dstc = dst * R + t
    b1r = b1.reshape(1, D)
    b2r = b2.reshape(1, D)

    cnt_p = _count_kernel(dstc)
    invc = _inv(cnt_p).reshape(NB)
    escale = _escale_kernel(dstc, invc)
    d3 = dst.reshape(NTILES * NPH, PCH, CH)
    y1, root1 = _mm1(x, W_rel1, W_root1, b1r)
    acc1 = _scatter_kernel(y1.reshape(R * N, D), src_row, d3,
                           escale).reshape(2, N, D)
    y2, root2 = _mm2(acc1, root1, W_rel2, W_root2, b2r)
    acc2 = _scatter_kernel(y2.reshape(R * N, D), src_row, d3,
                           escale).reshape(2, N, D)
    return _fin(acc2, root2)
